# baseline (speedup 1.0000x reference)
import functools

import jax
import jax.numpy as jnp
from jax import lax
from jax.experimental import pallas as pl
from jax.experimental.pallas import tpu as pltpu
from jax.experimental.pallas import tpu_sc as plsc

NC = 2
NS = 16
NW = NC * NS
CW = 128
NUM_GRAPHS = 64


def _sc_mesh():
    return plsc.VectorSubcoreMesh(core_axis_name="c", subcore_axis_name="s")


def _make_deg_kernel(npad, ept):
    rows = npad // NS
    n_full = ept // CW
    tail = ept % CW

    @functools.partial(
        pl.kernel,
        out_type=jax.ShapeDtypeStruct((NC, npad), jnp.float32),
        mesh=_sc_mesh(),
        scratch_types=[
            pltpu.VMEM((CW,), jnp.int32),
            pltpu.VMEM((CW,), jnp.int32),
            pltpu.VMEM((max(tail, 8),), jnp.int32),
            pltpu.VMEM((CW,), jnp.float32),
            pltpu.VMEM((rows,), jnp.float32),
            pltpu.SemaphoreType.DMA,
            pltpu.SemaphoreType.DMA,
            pltpu.SemaphoreType.DMA,
            pltpu.VMEM_SHARED((npad,), jnp.float32),
        ],
    )
    def deg_kernel(src_hbm, dst_hbm, deg_out, r0, r1, rt, ones_v, zero_v, i0, i1,
                   hsem, deg_sh):
        cid = lax.axis_index("c")
        sid = lax.axis_index("s")
        wid = sid * NC + cid
        base = wid * ept
        rng = (r0, r1)
        isem = (i0, i1)
        for j in range(CW // 16):
            ones_v[pl.ds(j * 16, 16)] = jnp.ones((16,), jnp.float32)

        def zfill(j, carry):
            zero_v[pl.ds(j * 16, 16)] = jnp.zeros((16,), jnp.float32)
            return carry

        lax.fori_loop(0, rows // 16, zfill, 0)
        pltpu.sync_copy(zero_v, deg_sh.at[pl.ds(sid * rows, rows)])
        plsc.subcore_barrier()

        def chunk(i, carry):
            for b in range(2):
                pltpu.async_copy(
                    dst_hbm.at[pl.ds(base + (i * 2 + b) * CW, CW)],
                    rng[b], isem[b])
            for b in range(2):
                pltpu.make_async_copy(
                    dst_hbm.at[pl.ds(base, CW)], rng[b], isem[b]).wait()
                pltpu.async_copy(ones_v, deg_sh.at[rng[b]], hsem, add=True)
            for b in range(2):
                pltpu.make_async_copy(ones_v, deg_sh.at[rng[0]],
                                      hsem).wait()
            return carry

        lax.fori_loop(0, n_full // 2, chunk, 0)
        if n_full % 2:
            c = n_full - 1
            pltpu.sync_copy(dst_hbm.at[pl.ds(base + c * CW, CW)], rng[0])
            pltpu.sync_copy(ones_v, deg_sh.at[rng[0]], add=True)
        if tail:
            pltpu.sync_copy(
                dst_hbm.at[pl.ds(base + n_full * CW, tail)], rt)
            pltpu.sync_copy(ones_v.at[pl.ds(0, tail)],
                            deg_sh.at[rt], add=True)
        plsc.subcore_barrier()
        pltpu.sync_copy(deg_sh.at[pl.ds(sid * rows, rows)],
                        deg_out.at[cid, pl.ds(sid * rows, rows)])

    return deg_kernel


def _make_edge_kernel(npad, ept, d):
    rows = npad // NS
    n_full = ept // CW
    tail = ept % CW

    @functools.partial(
        pl.kernel,
        out_type=jax.ShapeDtypeStruct((NC, npad, d), jnp.float32),
        mesh=_sc_mesh(),
        scratch_types=[
            pltpu.VMEM((CW,), jnp.int32),
            pltpu.VMEM((CW,), jnp.int32),
            pltpu.VMEM((CW,), jnp.int32),
            pltpu.VMEM((CW,), jnp.int32),
            pltpu.VMEM((max(tail, 8),), jnp.int32),
            pltpu.VMEM((max(tail, 8),), jnp.int32),
            pltpu.VMEM((CW, d), jnp.float32),
            pltpu.VMEM((CW, d), jnp.float32),
            pltpu.VMEM((max(tail, 8), d), jnp.float32),
            [pltpu.SemaphoreType.DMA] * 8,
            pltpu.VMEM_SHARED((npad, d), jnp.float32),
        ],
    )
    def edge_kernel(src_hbm, dst_hbm, y_hbm, zblk_hbm, out_hbm,
                    sr0, sr1, d0, d1, st_, dt_, b0, b1, bt, sems, out_sh):
        cid = lax.axis_index("c")
        sid = lax.axis_index("s")
        wid = sid * NC + cid
        base = wid * ept
        srng = (sr0, sr1)
        drng = (d0, d1)
        bufs = (b0, b1)
        gsem = sems[0:2]
        ssem = sems[2:4]
        psem = sems[4:6]
        dsem = sems[6:8]
        pltpu.sync_copy(zblk_hbm, out_sh.at[pl.ds(sid * rows, rows)])
        plsc.subcore_barrier()

        pltpu.async_copy(src_hbm.at[pl.ds(base, CW)], srng[0], psem[0])
        pltpu.async_copy(src_hbm.at[pl.ds(base + CW, CW)], srng[1],
                         psem[1])
        pltpu.async_copy(dst_hbm.at[pl.ds(base, CW)], drng[0], dsem[0])
        pltpu.make_async_copy(
            src_hbm.at[pl.ds(base, CW)], srng[0], psem[0]).wait()
        pltpu.async_copy(y_hbm.at[srng[0]], bufs[0], gsem[0])

        def group(i, carry):
            for b in range(2):
                c = i * 2 + b
                nb = 1 - b
                pltpu.make_async_copy(
                    y_hbm.at[srng[b]], bufs[b], gsem[b]).wait()

                @pl.when(c >= 1)
                def _():
                    pltpu.make_async_copy(
                        bufs[nb], out_sh.at[drng[nb]], ssem[nb]).wait()

                @pl.when(c + 1 < n_full)
                def _():
                    pltpu.make_async_copy(
                        src_hbm.at[pl.ds(base, CW)], srng[nb],
                        psem[nb]).wait()
                    pltpu.async_copy(y_hbm.at[srng[nb]], bufs[nb],
                                     gsem[nb])
                    pltpu.async_copy(
                        dst_hbm.at[pl.ds(base + (c + 1) * CW, CW)],
                        drng[nb], dsem[nb])

                @pl.when(c + 2 < n_full)
                def _():
                    pltpu.async_copy(
                        src_hbm.at[pl.ds(base + (c + 2) * CW, CW)],
                        srng[b], psem[b])

                pltpu.make_async_copy(
                    dst_hbm.at[pl.ds(base, CW)], drng[b], dsem[b]).wait()
                pltpu.async_copy(bufs[b], out_sh.at[drng[b]], ssem[b],
                                 add=True)
            return carry

        lax.fori_loop(0, n_full // 2, group, 0)
        if n_full % 2:
            bl = (n_full - 1) % 2
            pltpu.make_async_copy(
                y_hbm.at[srng[bl]], bufs[bl], gsem[bl]).wait()
            pltpu.make_async_copy(
                bufs[1 - bl], out_sh.at[drng[1 - bl]], ssem[1 - bl]).wait()
            pltpu.make_async_copy(
                dst_hbm.at[pl.ds(base, CW)], drng[bl], dsem[bl]).wait()
            pltpu.sync_copy(bufs[bl], out_sh.at[drng[bl]], add=True)
        else:
            pltpu.make_async_copy(
                bufs[(n_full - 1) % 2], out_sh.at[drng[(n_full - 1) % 2]],
                ssem[(n_full - 1) % 2]).wait()
        if tail:
            toff = base + n_full * CW
            pltpu.sync_copy(src_hbm.at[pl.ds(toff, tail)], st_)
            pltpu.sync_copy(dst_hbm.at[pl.ds(toff, tail)], dt_)
            pltpu.sync_copy(y_hbm.at[st_], bt)
            pltpu.sync_copy(bt, out_sh.at[dt_], add=True)
        plsc.subcore_barrier()
        pltpu.sync_copy(out_sh.at[pl.ds(sid * rows, rows)],
                        out_hbm.at[cid, pl.ds(sid * rows, rows)])

    return edge_kernel


def _y_body(n_valid, npad, deg_ref, x_ref, w_ref, y_ref):
    d = deg_ref[pl.ds(0, n_valid), 0:1] + deg_ref[pl.ds(0, n_valid), 1:2]
    dis = lax.rsqrt(d + 1.0)
    xw = jnp.dot(x_ref[...], w_ref[...], preferred_element_type=jnp.float32)
    y_ref[pl.ds(0, n_valid), :] = xw * dis
    y_ref[pl.ds(n_valid, npad - n_valid), :] = jnp.zeros(
        (npad - n_valid, xw.shape[1]), jnp.float32)


def _pool_body(n_valid, npad, deg_ref, p_ref, y_ref, b2_ref, batch_ref,
               lw_ref, lb_ref, o_ref):
    d = deg_ref[:, 0:1] + deg_ref[:, 1:2] + 1.0
    dis = lax.rsqrt(d)
    valid = lax.broadcasted_iota(jnp.int32, (npad, 1), 0) < n_valid
    dis = jnp.where(valid, dis, 0.0)
    acc = p_ref[0] + p_ref[1] + y_ref[...]
    h = jnp.maximum(acc * dis + b2_ref[...], 0.0)
    gids = lax.broadcasted_iota(jnp.int32, (npad, NUM_GRAPHS), 1)
    onehot = (batch_ref[...] == gids).astype(jnp.float32)
    sums = lax.dot_general(onehot, h, (((0,), (0,)), ((), ())),
                           preferred_element_type=jnp.float32)
    counts = lax.dot_general(onehot, jnp.ones((npad, 1), jnp.float32),
                             (((0,), (0,)), ((), ())),
                             preferred_element_type=jnp.float32)
    pooled = sums / jnp.maximum(counts, 1.0)
    out = lax.dot_general(pooled, lw_ref[...], (((1,), (1,)), ((), ())),
                          preferred_element_type=jnp.float32)
    o_ref[...] = out + lb_ref[...]


def kernel(x, edge_index, batch, W1, b1, W2, b2, lin_W, lin_b):
    n, d_feat = x.shape
    d = W2.shape[1]
    e = edge_index.shape[1]
    npad = -(-n // 256) * 256
    if npad == n:
        npad += 256

    ei = edge_index.astype(jnp.int32)
    if e % (NW * 8):
        pad_e = NW * 8 - e % (NW * 8)
        dummy = n + (jnp.arange(pad_e, dtype=jnp.int32) % (npad - n))
        ei = jnp.concatenate([ei, jnp.stack([dummy, dummy])], axis=1)
    ept = ei.shape[1] // NW

    src1 = ei[0]
    dst1 = ei[1]
    deg2 = _make_deg_kernel(npad, ept)(src1, dst1)
    deg_t = deg2.T

    y = pl.pallas_call(
        functools.partial(_y_body, n, npad),
        out_shape=jax.ShapeDtypeStruct((npad, d), jnp.float32),
    )(deg_t, x, W2)

    zblk = jnp.zeros((npad // NS, d), jnp.float32)
    p = _make_edge_kernel(npad, ept, d)(src1, dst1, y, zblk)

    batch_col = jnp.pad(batch.astype(jnp.int32), (0, npad - n),
                        constant_values=-1).reshape(npad, 1)
    out = pl.pallas_call(
        functools.partial(_pool_body, n, npad),
        out_shape=jax.ShapeDtypeStruct((NUM_GRAPHS, d), jnp.float32),
    )(deg_t, p, y, b2.reshape(1, -1), batch_col, lin_W, lin_b.reshape(1, -1))
    return out


# speedup vs baseline: 1.1281x; 1.1281x over previous
import functools

import jax
import jax.numpy as jnp
from jax import lax
from jax.experimental import pallas as pl
from jax.experimental.pallas import tpu as pltpu
from jax.experimental.pallas import tpu_sc as plsc

NC = 2
NS = 16
NW = NC * NS
CW = 128
NUM_GRAPHS = 64


def _sc_mesh():
    return plsc.VectorSubcoreMesh(core_axis_name="c", subcore_axis_name="s")


def _make_deg_kernel(npad, nch):
    rows = npad // NS

    @functools.partial(
        pl.kernel,
        out_type=jax.ShapeDtypeStruct((NC, npad), jnp.float32),
        mesh=_sc_mesh(),
        scratch_types=[
            pltpu.VMEM((nch, CW), jnp.int32),
            pltpu.VMEM((CW,), jnp.float32),
            pltpu.VMEM((rows,), jnp.float32),
            pltpu.SemaphoreType.DMA,
            pltpu.VMEM_SHARED((npad,), jnp.float32),
        ],
    )
    def deg_kernel(dst_hbm, deg_out, idx_v, ones_v, zero_v, hsem, deg_sh):
        cid = lax.axis_index("c")
        sid = lax.axis_index("s")
        wid = sid * NC + cid
        for j in range(CW // 16):
            ones_v[pl.ds(j * 16, 16)] = jnp.ones((16,), jnp.float32)

        def zfill(j, carry):
            zero_v[pl.ds(j * 16, 16)] = jnp.zeros((16,), jnp.float32)
            return carry

        lax.fori_loop(0, rows // 16, zfill, 0)
        pltpu.sync_copy(zero_v, deg_sh.at[pl.ds(sid * rows, rows)])
        pltpu.sync_copy(dst_hbm.at[wid], idx_v)
        plsc.subcore_barrier()

        def chunk(i, carry):
            for b in range(4):
                pltpu.async_copy(ones_v, deg_sh.at[idx_v.at[i * 4 + b]],
                                 hsem, add=True)
            for b in range(4):
                pltpu.make_async_copy(ones_v, deg_sh.at[idx_v.at[0]],
                                      hsem).wait()
            return carry

        lax.fori_loop(0, nch // 4, chunk, 0)
        plsc.subcore_barrier()
        pltpu.sync_copy(deg_sh.at[pl.ds(sid * rows, rows)],
                        deg_out.at[cid, pl.ds(sid * rows, rows)])

    return deg_kernel


def _make_edge_kernel(npad, nch, d):
    rows = npad // NS

    @functools.partial(
        pl.kernel,
        out_type=jax.ShapeDtypeStruct((NC, npad, d), jnp.float32),
        mesh=_sc_mesh(),
        scratch_types=[
            pltpu.VMEM((nch, CW), jnp.int32),
            pltpu.VMEM((CW,), jnp.int32),
            pltpu.VMEM((CW,), jnp.int32),
            pltpu.VMEM((CW, d), jnp.float32),
            pltpu.VMEM((CW, d), jnp.float32),
            pltpu.VMEM((32, d), jnp.float32),
            pltpu.SemaphoreType.DMA,
            pltpu.SemaphoreType.DMA,
            pltpu.SemaphoreType.DMA,
            pltpu.SemaphoreType.DMA,
            pltpu.SemaphoreType.DMA,
            pltpu.SemaphoreType.DMA,
            pltpu.SemaphoreType.DMA,
            pltpu.VMEM_SHARED((npad, d), jnp.float32),
        ],
    )
    def edge_kernel(src_hbm, dst_hbm, y_hbm, out_hbm,
                    srcv, d0, d1, b0, b1, zv, g0, g1, t0, t1, s0, s1, zs,
                    out_sh):
        cid = lax.axis_index("c")
        sid = lax.axis_index("s")
        wid = sid * NC + cid
        bufs = (b0, b1)
        gsem = (g0, g1)
        drng = (d0, d1)
        dsem = (t0, t1)
        ssem = (s0, s1)
        for r in range(32):
            for j in range(d // 16):
                zv[r, pl.ds(j * 16, 16)] = jnp.zeros((16,), jnp.float32)
        for k in range(rows // 32):
            pltpu.async_copy(zv, out_sh.at[pl.ds(sid * rows + k * 32, 32)],
                             zs)
        pltpu.sync_copy(src_hbm.at[wid], srcv)
        for k in range(rows // 32):
            pltpu.make_async_copy(
                zv, out_sh.at[pl.ds(sid * rows, 32)], zs).wait()
        plsc.subcore_barrier()

        pltpu.async_copy(y_hbm.at[srcv.at[0]], bufs[0], gsem[0])
        pltpu.async_copy(dst_hbm.at[wid, 0], drng[0], dsem[0])

        def group(i, carry):
            for b in range(2):
                c = i * 2 + b
                nb = 1 - b
                pltpu.make_async_copy(
                    y_hbm.at[srcv.at[c]], bufs[b], gsem[b]).wait()

                @pl.when(c >= 1)
                def _():
                    pltpu.make_async_copy(
                        bufs[nb], out_sh.at[drng[nb]], ssem[nb]).wait()

                @pl.when(c + 1 < nch)
                def _():
                    pltpu.async_copy(
                        y_hbm.at[srcv.at[c + 1]], bufs[nb], gsem[nb])
                    pltpu.async_copy(
                        dst_hbm.at[wid, c + 1], drng[nb], dsem[nb])

                pltpu.make_async_copy(
                    dst_hbm.at[wid, c], drng[b], dsem[b]).wait()
                pltpu.async_copy(bufs[b], out_sh.at[drng[b]], ssem[b],
                                 add=True)
            return carry

        lax.fori_loop(0, nch // 2, group, 0)
        pltpu.make_async_copy(
            bufs[(nch - 1) % 2], out_sh.at[drng[(nch - 1) % 2]],
            ssem[(nch - 1) % 2]).wait()
        plsc.subcore_barrier()
        pltpu.sync_copy(out_sh.at[pl.ds(sid * rows, rows)],
                        out_hbm.at[cid, pl.ds(sid * rows, rows)])

    return edge_kernel


def _y_body(n_valid, npad, deg_ref, x_ref, w_ref, y_ref):
    d = deg_ref[pl.ds(0, n_valid), 0:1] + deg_ref[pl.ds(0, n_valid), 1:2]
    dis = lax.rsqrt(d + 1.0)
    xw = jnp.dot(x_ref[...], w_ref[...], preferred_element_type=jnp.float32)
    y_ref[pl.ds(0, n_valid), :] = xw * dis
    y_ref[pl.ds(n_valid, npad - n_valid), :] = jnp.zeros(
        (npad - n_valid, xw.shape[1]), jnp.float32)


def _pool_body(n_valid, npad, deg_ref, p_ref, y_ref, b2_ref, batch_ref,
               lw_ref, lb_ref, o_ref):
    d = deg_ref[:, 0:1] + deg_ref[:, 1:2] + 1.0
    dis = lax.rsqrt(d)
    valid = lax.broadcasted_iota(jnp.int32, (npad, 1), 0) < n_valid
    dis = jnp.where(valid, dis, 0.0)
    acc = p_ref[0] + p_ref[1] + y_ref[...]
    h = jnp.maximum(acc * dis + b2_ref[...], 0.0)
    gids = lax.broadcasted_iota(jnp.int32, (npad, NUM_GRAPHS), 1)
    onehot = (batch_ref[...] == gids).astype(jnp.float32)
    sums = lax.dot_general(onehot, h, (((0,), (0,)), ((), ())),
                           preferred_element_type=jnp.float32)
    counts = lax.dot_general(onehot, jnp.ones((npad, 1), jnp.float32),
                             (((0,), (0,)), ((), ())),
                             preferred_element_type=jnp.float32)
    pooled = sums / jnp.maximum(counts, 1.0)
    out = lax.dot_general(pooled, lw_ref[...], (((1,), (1,)), ((), ())),
                          preferred_element_type=jnp.float32)
    o_ref[...] = out + lb_ref[...]


def kernel(x, edge_index, batch, W1, b1, W2, b2, lin_W, lin_b):
    n, d_feat = x.shape
    d = W2.shape[1]
    e = edge_index.shape[1]
    npad = -(-n // 256) * 256
    if npad == n:
        npad += 256
    nch = -(-e // (NW * CW))
    nch = -(-nch // 4) * 4
    e_pad = NW * nch * CW

    src = edge_index[0].astype(jnp.int32)
    dst = edge_index[1].astype(jnp.int32)
    dummy = n + (jnp.arange(e_pad - e, dtype=jnp.int32) % (npad - n))
    src3 = jnp.concatenate([src, dummy]).reshape(NW, nch, CW)
    dst3 = jnp.concatenate([dst, dummy]).reshape(NW, nch, CW)

    deg2 = _make_deg_kernel(npad, nch)(dst3)
    deg_t = deg2.T

    y = pl.pallas_call(
        functools.partial(_y_body, n, npad),
        out_shape=jax.ShapeDtypeStruct((npad, d), jnp.float32),
    )(deg_t, x, W2)

    p = _make_edge_kernel(npad, nch, d)(src3, dst3, y)

    batch_col = jnp.pad(batch.astype(jnp.int32), (0, npad - n),
                        constant_values=-1).reshape(npad, 1)
    out = pl.pallas_call(
        functools.partial(_pool_body, n, npad),
        out_shape=jax.ShapeDtypeStruct((NUM_GRAPHS, d), jnp.float32),
    )(deg_t, p, y, b2.reshape(1, -1), batch_col, lin_W, lin_b.reshape(1, -1))
    return out
